# baseline (speedup 1.0000x reference)
import functools

import jax
import jax.numpy as jnp
from jax import lax
from jax.experimental import pallas as pl
from jax.experimental.pallas import tpu as pltpu
from jax.experimental.pallas import tpu_sc as plsc

N = 10000
H = 128
D_IN = 256
OUT = 64
E = 160000
ALPHA = 0.5

NCORE = 2
NSUB = 16
NW = NCORE * NSUB
CH = 64
NCH0 = 48
NCH1 = 112
NCHMAX = max(NCH0, NCH1)
GR = 16
NCHD = 80
NPAD = 10112
SLICE = NPAD // NSUB
NBUF = 3
DW = 16

_mesh = plsc.VectorSubcoreMesh(core_axis_name="c", subcore_axis_name="s")


@functools.partial(
    pl.kernel,
    mesh=_mesh,
    out_type=jax.ShapeDtypeStruct((NCORE, NPAD, DW), jnp.float32),
    scratch_types=[
        pltpu.VMEM((NCHD // 2, CH), jnp.int32),
        pltpu.VMEM((CH, DW), jnp.float32),
        pltpu.VMEM_SHARED((NPAD, DW), jnp.float32),
    ],
)
def _sc_deg(dst3, deg_out, dst_v, ones_v, acc_sp):
    c = lax.axis_index("c")
    s = lax.axis_index("s")
    w = c * NSUB + s

    def zero_body(i, _):
        ones_v[i, pl.ds(0, DW)] = jnp.zeros((DW,), jnp.float32)
        return 0

    lax.fori_loop(0, CH, zero_body, 0)
    for off in range(0, SLICE, CH):
        nr = min(CH, SLICE - off)
        pltpu.sync_copy(ones_v.at[pl.ds(0, nr)],
                        acc_sp.at[pl.ds(s * SLICE + off, nr)])

    def ones_body(i, _):
        ones_v[i, pl.ds(0, DW)] = jnp.ones((DW,), jnp.float32)
        return 0

    lax.fori_loop(0, CH, ones_body, 0)
    plsc.subcore_barrier()

    def chunk_body(k, _):
        pltpu.sync_copy(ones_v, acc_sp.at[dst_v.at[k]], add=True)
        return 0

    for half in range(2):
        pltpu.sync_copy(
            dst3.at[w, pl.ds(half * (NCHD // 2), NCHD // 2)], dst_v)
        lax.fori_loop(0, NCHD // 2, chunk_body, 0)
    plsc.subcore_barrier()
    pltpu.sync_copy(
        acc_sp.at[pl.ds(s * SLICE, SLICE)],
        deg_out.at[c, pl.ds(s * SLICE, SLICE)],
    )


@functools.partial(
    pl.kernel,
    mesh=_mesh,
    out_type=jax.ShapeDtypeStruct((NCORE, NPAD, H), jnp.float32),
    scratch_types=[
        pltpu.VMEM((2, GR, CH), jnp.int32),
        pltpu.VMEM((2, GR, CH), jnp.int32),
        pltpu.VMEM((NBUF, CH, H), jnp.float32),
        pltpu.VMEM_SHARED((NPAD, H), jnp.float32),
        pltpu.SemaphoreType.DMA,
        pltpu.SemaphoreType.DMA,
    ] + [pltpu.SemaphoreType.DMA] * NBUF,
)
def _sc_prop(xs, src3, dst3, out, src_v, dst_v, rows_v, acc_sp,
             isem1, isem2, *gsem):
    c = lax.axis_index("c")
    s = lax.axis_index("s")
    w = c * NSUB + s

    def zero_body(i, _):
        for j in range(H // 16):
            rows_v[0, i, pl.ds(j * 16, 16)] = jnp.zeros((16,), jnp.float32)
        return 0

    lax.fori_loop(0, CH, zero_body, 0)
    for off in range(0, SLICE, CH):
        nr = min(CH, SLICE - off)
        pltpu.sync_copy(rows_v.at[0, pl.ds(0, nr)],
                        acc_sp.at[pl.ds(s * SLICE + off, nr)])
    plsc.subcore_barrier()

    nch = jnp.where(c == 0, NCH0, NCH1)
    ng = nch // GR

    @pl.when(nch > 0)
    def _():
        pltpu.sync_copy(src3.at[w, pl.ds(0, GR)], src_v.at[0])
        pltpu.sync_copy(dst3.at[w, pl.ds(0, GR)], dst_v.at[0])
        for b in range(NBUF):
            pltpu.async_copy(xs.at[src_v.at[0, b]], rows_v.at[b], gsem[b])

    def group_body(g, _):
        gb = lax.rem(g, 2)
        ngb = 1 - gb

        @pl.when(g + 1 < ng)
        def _():
            pltpu.async_copy(
                src3.at[w, pl.ds((g + 1) * GR, GR)], src_v.at[ngb], isem1)
            pltpu.async_copy(
                dst3.at[w, pl.ds((g + 1) * GR, GR)], dst_v.at[ngb], isem2)

        for j in range(GR):
            b = j % NBUF
            k = g * GR + j
            pltpu.make_async_copy(
                xs.at[src_v.at[gb, j]], rows_v.at[b], gsem[b]).wait()
            pltpu.sync_copy(
                rows_v.at[b], acc_sp.at[dst_v.at[gb, j]], add=True)
            jn = j + NBUF
            if jn < GR:
                @pl.when(k + NBUF < nch)
                def _():
                    pltpu.async_copy(
                        xs.at[src_v.at[gb, jn]], rows_v.at[b], gsem[b])
            else:
                if jn - GR == 0:
                    @pl.when(g + 1 < ng)
                    def _():
                        pltpu.make_async_copy(
                            src3.at[w, pl.ds(0, GR)], src_v.at[ngb],
                            isem1).wait()
                        pltpu.make_async_copy(
                            dst3.at[w, pl.ds(0, GR)], dst_v.at[ngb],
                            isem2).wait()

                @pl.when(g + 1 < ng)
                def _():
                    pltpu.async_copy(
                        xs.at[src_v.at[ngb, jn - GR]], rows_v.at[b], gsem[b])
        return 0

    lax.fori_loop(0, ng, group_body, 0)
    plsc.subcore_barrier()
    for off in range(0, SLICE, CH):
        nr = min(CH, SLICE - off)
        pltpu.sync_copy(acc_sp.at[pl.ds(s * SLICE + off, nr)],
                        out.at[c, pl.ds(s * SLICE + off, nr)])


def _dinv(deg_ref):
    deg = deg_ref[0, 0:N, 0:1] + deg_ref[1, 0:N, 0:1] + 1.0
    return lax.rsqrt(deg)


def _layer_norm(t, g, b):
    mu = jnp.mean(t, axis=-1, keepdims=True)
    var = jnp.mean((t - mu) ** 2, axis=-1, keepdims=True)
    return (t - mu) * lax.rsqrt(var + 1e-5) * g + b


def _tc1_body(x_ref, w_ref, b_ref, deg_ref, x0_ref, xs0_ref):
    x0 = jnp.maximum(
        jnp.dot(x_ref[...], w_ref[...], preferred_element_type=jnp.float32)
        + b_ref[...],
        0.0,
    )
    x0_ref[...] = x0
    xs0_ref[...] = x0 * _dinv(deg_ref)


def _tc2_body(pa_ref, xs0_ref, x0_ref, deg_ref, w1_ref, g1_ref, be1_ref,
              xs1_ref):
    dinv = _dinv(deg_ref)
    acc = pa_ref[0, 0:N, :] + pa_ref[1, 0:N, :]
    h = dinv * (acc + xs0_ref[...])
    mix = (1.0 - ALPHA) * h + ALPHA * x0_ref[...]
    t = jnp.dot(mix, w1_ref[...], preferred_element_type=jnp.float32)
    h1 = jnp.maximum(_layer_norm(t, g1_ref[...], be1_ref[...]), 0.0)
    xs1_ref[...] = h1 * dinv


def _tc3_body(pb_ref, xs1_ref, x0_ref, deg_ref, w2_ref, g2_ref, be2_ref,
              wr1_ref, br1_ref, gr_ref, ber_ref, wr2_ref, br2_ref, out_ref):
    dinv = _dinv(deg_ref)
    acc = pb_ref[0, 0:N, :] + pb_ref[1, 0:N, :]
    h = dinv * (acc + xs1_ref[...])
    mix = (1.0 - ALPHA) * h + ALPHA * x0_ref[...]
    t = jnp.dot(mix, w2_ref[...], preferred_element_type=jnp.float32)
    h2 = jnp.maximum(_layer_norm(t, g2_ref[...], be2_ref[...]), 0.0)
    zr = jnp.dot(h2, wr1_ref[...], preferred_element_type=jnp.float32) \
        + br1_ref[...]
    z = jnp.maximum(_layer_norm(zr, gr_ref[...], ber_ref[...]), 0.0)
    out_ref[...] = jnp.dot(
        z, wr2_ref[...], preferred_element_type=jnp.float32
    ) + br2_ref[...]


_tc1 = pl.pallas_call(
    _tc1_body,
    out_shape=(
        jax.ShapeDtypeStruct((N, H), jnp.float32),
        jax.ShapeDtypeStruct((N, H), jnp.float32),
    ),
)

_tc2 = pl.pallas_call(
    _tc2_body,
    out_shape=jax.ShapeDtypeStruct((N, H), jnp.float32),
)

_tc3 = pl.pallas_call(
    _tc3_body,
    out_shape=jax.ShapeDtypeStruct((N, OUT), jnp.float32),
)


def kernel(x, edge_index, edge_weight, W_in, b_in, W1, g1, be1, W2, g2, be2,
           Wr1, br1, gr, ber, Wr2, br2):
    src = edge_index[0]
    dst = edge_index[1]

    capd = NW * NCHD * CH
    dstd = jnp.concatenate(
        [dst, jnp.full((capd - E,), N, jnp.int32)]).reshape(NW, NCHD, CH)

    cap0 = NSUB * NCH0 * CH
    cap1 = NSUB * NCH1 * CH

    def split3(e, fill):
        ep = jnp.concatenate([e, jnp.full((cap0 + cap1 - E,), fill, e.dtype)])
        p0 = jnp.pad(ep[:cap0].reshape(NSUB, NCH0, CH),
                     ((0, 0), (0, NCHMAX - NCH0), (0, 0)),
                     constant_values=fill)
        p1 = jnp.pad(ep[cap0:].reshape(NSUB, NCH1, CH),
                     ((0, 0), (0, NCHMAX - NCH1), (0, 0)),
                     constant_values=fill)
        return jnp.concatenate([p0, p1], axis=0)

    src3 = split3(src, 0)
    dst3 = split3(dst, N)

    deg3 = _sc_deg(dstd)

    x0, xs0 = _tc1(x, W_in, b_in.reshape(1, H), deg3)
    pa = _sc_prop(xs0, src3, dst3)
    xs1 = _tc2(pa, xs0, x0, deg3, W1, g1.reshape(1, H), be1.reshape(1, H))
    pb = _sc_prop(xs1, src3, dst3)
    return _tc3(pb, xs1, x0, deg3, W2, g2.reshape(1, H), be2.reshape(1, H),
                Wr1, br1.reshape(1, 32), gr.reshape(1, 32),
                ber.reshape(1, 32), Wr2, br2.reshape(1, OUT))


# speedup vs baseline: 1.0117x; 1.0117x over previous
import functools

import jax
import jax.numpy as jnp
from jax import lax
from jax.experimental import pallas as pl
from jax.experimental.pallas import tpu as pltpu
from jax.experimental.pallas import tpu_sc as plsc

N = 10000
H = 128
D_IN = 256
OUT = 64
E = 160000
ALPHA = 0.5

NCORE = 2
NSUB = 16
NW = NCORE * NSUB
CH = 64
NCH0 = 48
NCH1 = 112
NCHMAX = max(NCH0, NCH1)
GR = 8
NCHD = 80
NPAD = 10112
SLICE = NPAD // NSUB
NBUF = 4
DW = 16

_mesh = plsc.VectorSubcoreMesh(core_axis_name="c", subcore_axis_name="s")


@functools.partial(
    pl.kernel,
    mesh=_mesh,
    out_type=jax.ShapeDtypeStruct((NCORE, NPAD, DW), jnp.float32),
    scratch_types=[
        pltpu.VMEM((NCHD // 2, CH), jnp.int32),
        pltpu.VMEM((CH, DW), jnp.float32),
        pltpu.VMEM_SHARED((NPAD, DW), jnp.float32),
    ],
)
def _sc_deg(dst3, deg_out, dst_v, ones_v, acc_sp):
    c = lax.axis_index("c")
    s = lax.axis_index("s")
    w = c * NSUB + s

    def zero_body(i, _):
        ones_v[i, pl.ds(0, DW)] = jnp.zeros((DW,), jnp.float32)
        return 0

    lax.fori_loop(0, CH, zero_body, 0)
    for off in range(0, SLICE, CH):
        nr = min(CH, SLICE - off)
        pltpu.sync_copy(ones_v.at[pl.ds(0, nr)],
                        acc_sp.at[pl.ds(s * SLICE + off, nr)])

    def ones_body(i, _):
        ones_v[i, pl.ds(0, DW)] = jnp.ones((DW,), jnp.float32)
        return 0

    lax.fori_loop(0, CH, ones_body, 0)
    plsc.subcore_barrier()

    def chunk_body(k, _):
        pltpu.sync_copy(ones_v, acc_sp.at[dst_v.at[k]], add=True)
        return 0

    for half in range(2):
        pltpu.sync_copy(
            dst3.at[w, pl.ds(half * (NCHD // 2), NCHD // 2)], dst_v)
        lax.fori_loop(0, NCHD // 2, chunk_body, 0)
    plsc.subcore_barrier()
    pltpu.sync_copy(
        acc_sp.at[pl.ds(s * SLICE, SLICE)],
        deg_out.at[c, pl.ds(s * SLICE, SLICE)],
    )


@functools.partial(
    pl.kernel,
    mesh=_mesh,
    out_type=jax.ShapeDtypeStruct((NCORE, NPAD, H), jnp.float32),
    scratch_types=[
        pltpu.VMEM((2, GR, CH), jnp.int32),
        pltpu.VMEM((2, GR, CH), jnp.int32),
        pltpu.VMEM((NBUF, CH, H), jnp.float32),
        pltpu.VMEM_SHARED((NPAD, H), jnp.float32),
        pltpu.SemaphoreType.DMA,
        pltpu.SemaphoreType.DMA,
    ] + [pltpu.SemaphoreType.DMA] * NBUF,
)
def _sc_prop(xs, src3, dst3, out, src_v, dst_v, rows_v, acc_sp,
             isem1, isem2, *gsem):
    c = lax.axis_index("c")
    s = lax.axis_index("s")
    w = c * NSUB + s

    def zero_body(i, _):
        for j in range(H // 16):
            rows_v[0, i, pl.ds(j * 16, 16)] = jnp.zeros((16,), jnp.float32)
        return 0

    lax.fori_loop(0, CH, zero_body, 0)
    for off in range(0, SLICE, CH):
        nr = min(CH, SLICE - off)
        pltpu.sync_copy(rows_v.at[0, pl.ds(0, nr)],
                        acc_sp.at[pl.ds(s * SLICE + off, nr)])
    plsc.subcore_barrier()

    nch = jnp.where(c == 0, NCH0, NCH1)
    ng = nch // GR

    @pl.when(nch > 0)
    def _():
        pltpu.sync_copy(src3.at[w, pl.ds(0, GR)], src_v.at[0])
        pltpu.sync_copy(dst3.at[w, pl.ds(0, GR)], dst_v.at[0])
        for b in range(NBUF):
            pltpu.async_copy(xs.at[src_v.at[0, b]], rows_v.at[b], gsem[b])

    def group_body(g, _):
        gb = lax.rem(g, 2)
        ngb = 1 - gb

        @pl.when(g + 1 < ng)
        def _():
            pltpu.async_copy(
                src3.at[w, pl.ds((g + 1) * GR, GR)], src_v.at[ngb], isem1)
            pltpu.async_copy(
                dst3.at[w, pl.ds((g + 1) * GR, GR)], dst_v.at[ngb], isem2)

        for j in range(GR):
            b = j % NBUF
            k = g * GR + j
            pltpu.make_async_copy(
                xs.at[src_v.at[gb, j]], rows_v.at[b], gsem[b]).wait()
            pltpu.sync_copy(
                rows_v.at[b], acc_sp.at[dst_v.at[gb, j]], add=True)
            jn = j + NBUF
            if jn < GR:
                @pl.when(k + NBUF < nch)
                def _():
                    pltpu.async_copy(
                        xs.at[src_v.at[gb, jn]], rows_v.at[b], gsem[b])
            else:
                if jn - GR == 0:
                    @pl.when(g + 1 < ng)
                    def _():
                        pltpu.make_async_copy(
                            src3.at[w, pl.ds(0, GR)], src_v.at[ngb],
                            isem1).wait()
                        pltpu.make_async_copy(
                            dst3.at[w, pl.ds(0, GR)], dst_v.at[ngb],
                            isem2).wait()

                @pl.when(g + 1 < ng)
                def _():
                    pltpu.async_copy(
                        xs.at[src_v.at[ngb, jn - GR]], rows_v.at[b], gsem[b])
        return 0

    lax.fori_loop(0, ng, group_body, 0)
    plsc.subcore_barrier()
    for off in range(0, SLICE, CH):
        nr = min(CH, SLICE - off)
        pltpu.sync_copy(acc_sp.at[pl.ds(s * SLICE + off, nr)],
                        out.at[c, pl.ds(s * SLICE + off, nr)])


def _dinv(deg_ref):
    deg = deg_ref[0, 0:N, 0:1] + deg_ref[1, 0:N, 0:1] + 1.0
    return lax.rsqrt(deg)


def _layer_norm(t, g, b):
    mu = jnp.mean(t, axis=-1, keepdims=True)
    var = jnp.mean((t - mu) ** 2, axis=-1, keepdims=True)
    return (t - mu) * lax.rsqrt(var + 1e-5) * g + b


def _tc1_body(x_ref, w_ref, b_ref, deg_ref, x0_ref, xs0_ref):
    x0 = jnp.maximum(
        jnp.dot(x_ref[...], w_ref[...], preferred_element_type=jnp.float32)
        + b_ref[...],
        0.0,
    )
    x0_ref[...] = x0
    xs0_ref[...] = x0 * _dinv(deg_ref)


def _tc2_body(pa_ref, xs0_ref, x0_ref, deg_ref, w1_ref, g1_ref, be1_ref,
              xs1_ref):
    dinv = _dinv(deg_ref)
    acc = pa_ref[0, 0:N, :] + pa_ref[1, 0:N, :]
    h = dinv * (acc + xs0_ref[...])
    mix = (1.0 - ALPHA) * h + ALPHA * x0_ref[...]
    t = jnp.dot(mix, w1_ref[...], preferred_element_type=jnp.float32)
    h1 = jnp.maximum(_layer_norm(t, g1_ref[...], be1_ref[...]), 0.0)
    xs1_ref[...] = h1 * dinv


def _tc3_body(pb_ref, xs1_ref, x0_ref, deg_ref, w2_ref, g2_ref, be2_ref,
              wr1_ref, br1_ref, gr_ref, ber_ref, wr2_ref, br2_ref, out_ref):
    dinv = _dinv(deg_ref)
    acc = pb_ref[0, 0:N, :] + pb_ref[1, 0:N, :]
    h = dinv * (acc + xs1_ref[...])
    mix = (1.0 - ALPHA) * h + ALPHA * x0_ref[...]
    t = jnp.dot(mix, w2_ref[...], preferred_element_type=jnp.float32)
    h2 = jnp.maximum(_layer_norm(t, g2_ref[...], be2_ref[...]), 0.0)
    zr = jnp.dot(h2, wr1_ref[...], preferred_element_type=jnp.float32) \
        + br1_ref[...]
    z = jnp.maximum(_layer_norm(zr, gr_ref[...], ber_ref[...]), 0.0)
    out_ref[...] = jnp.dot(
        z, wr2_ref[...], preferred_element_type=jnp.float32
    ) + br2_ref[...]


_tc1 = pl.pallas_call(
    _tc1_body,
    out_shape=(
        jax.ShapeDtypeStruct((N, H), jnp.float32),
        jax.ShapeDtypeStruct((N, H), jnp.float32),
    ),
)

_tc2 = pl.pallas_call(
    _tc2_body,
    out_shape=jax.ShapeDtypeStruct((N, H), jnp.float32),
)

_tc3 = pl.pallas_call(
    _tc3_body,
    out_shape=jax.ShapeDtypeStruct((N, OUT), jnp.float32),
)


def kernel(x, edge_index, edge_weight, W_in, b_in, W1, g1, be1, W2, g2, be2,
           Wr1, br1, gr, ber, Wr2, br2):
    src = edge_index[0]
    dst = edge_index[1]

    capd = NW * NCHD * CH
    dstd = jnp.concatenate(
        [dst, jnp.full((capd - E,), N, jnp.int32)]).reshape(NW, NCHD, CH)

    cap0 = NSUB * NCH0 * CH
    cap1 = NSUB * NCH1 * CH

    def split3(e, fill):
        ep = jnp.concatenate([e, jnp.full((cap0 + cap1 - E,), fill, e.dtype)])
        p0 = jnp.pad(ep[:cap0].reshape(NSUB, NCH0, CH),
                     ((0, 0), (0, NCHMAX - NCH0), (0, 0)),
                     constant_values=fill)
        p1 = jnp.pad(ep[cap0:].reshape(NSUB, NCH1, CH),
                     ((0, 0), (0, NCHMAX - NCH1), (0, 0)),
                     constant_values=fill)
        return jnp.concatenate([p0, p1], axis=0)

    src3 = split3(src, 0)
    dst3 = split3(dst, N)

    deg3 = _sc_deg(dstd)

    x0, xs0 = _tc1(x, W_in, b_in.reshape(1, H), deg3)
    pa = _sc_prop(xs0, src3, dst3)
    xs1 = _tc2(pa, xs0, x0, deg3, W1, g1.reshape(1, H), be1.reshape(1, H))
    pb = _sc_prop(xs1, src3, dst3)
    return _tc3(pb, xs1, x0, deg3, W2, g2.reshape(1, H), be2.reshape(1, H),
                Wr1, br1.reshape(1, 32), gr.reshape(1, 32),
                ber.reshape(1, 32), Wr2, br2.reshape(1, OUT))


# speedup vs baseline: 1.5167x; 1.4991x over previous
import functools

import jax
import jax.numpy as jnp
from jax import lax
from jax.experimental import pallas as pl
from jax.experimental.pallas import tpu as pltpu
from jax.experimental.pallas import tpu_sc as plsc

N = 10000
H = 128
D_IN = 256
OUT = 64
E = 160000
ALPHA = 0.5

NCORE = 2
NSUB = 16
NW = NCORE * NSUB
CH = 64
NCH0 = 50
NCH1 = 108
NCHMAX = max(NCH0, NCH1)
GR = 8
NCHD = 80
NPAD = 10240
SLICE = NPAD // NSUB
NBUF = 2
DW = 16

_mesh = plsc.VectorSubcoreMesh(core_axis_name="c", subcore_axis_name="s")


@functools.partial(
    pl.kernel,
    mesh=_mesh,
    out_type=jax.ShapeDtypeStruct((NCORE, NPAD, DW), jnp.float32),
    scratch_types=[
        pltpu.VMEM((NCHD // 2, CH), jnp.int32),
        pltpu.VMEM((CH, DW), jnp.float32),
        pltpu.VMEM_SHARED((NPAD, DW), jnp.float32),
    ],
)
def _sc_deg(dst3, deg_out, dst_v, ones_v, acc_sp):
    c = lax.axis_index("c")
    s = lax.axis_index("s")
    w = c * NSUB + s

    def zero_body(i, _):
        ones_v[i, pl.ds(0, DW)] = jnp.zeros((DW,), jnp.float32)
        return 0

    lax.fori_loop(0, CH, zero_body, 0)
    for off in range(0, SLICE, CH):
        nr = min(CH, SLICE - off)
        pltpu.sync_copy(ones_v.at[pl.ds(0, nr)],
                        acc_sp.at[pl.ds(s * SLICE + off, nr)])

    def ones_body(i, _):
        ones_v[i, pl.ds(0, DW)] = jnp.ones((DW,), jnp.float32)
        return 0

    lax.fori_loop(0, CH, ones_body, 0)
    plsc.subcore_barrier()

    def chunk_body(k, _):
        pltpu.sync_copy(ones_v, acc_sp.at[dst_v.at[k]], add=True)
        return 0

    for half in range(2):
        pltpu.sync_copy(
            dst3.at[w, pl.ds(half * (NCHD // 2), NCHD // 2)], dst_v)
        lax.fori_loop(0, NCHD // 2, chunk_body, 0)
    plsc.subcore_barrier()
    pltpu.sync_copy(
        acc_sp.at[pl.ds(s * SLICE, SLICE)],
        deg_out.at[c, pl.ds(s * SLICE, SLICE)],
    )


@functools.partial(
    pl.kernel,
    mesh=_mesh,
    out_type=jax.ShapeDtypeStruct((NCORE, NPAD, H), jnp.float32),
    scratch_types=[
        pltpu.VMEM((NCHMAX, CH), jnp.int32),
        pltpu.VMEM((NCHMAX, CH), jnp.int32),
        pltpu.VMEM((NBUF, CH, H), jnp.float32),
        pltpu.VMEM_SHARED((NPAD, H), jnp.float32),
    ] + [pltpu.SemaphoreType.DMA] * NBUF,
)
def _sc_prop(xs, src3, dst3, out, src_v, dst_v, rows_v, acc_sp, *gsem):
    c = lax.axis_index("c")
    s = lax.axis_index("s")
    w = c * NSUB + s
    pltpu.sync_copy(src3.at[w], src_v)
    pltpu.sync_copy(dst3.at[w], dst_v)

    def zero_body(i, _):
        for j in range(H // 16):
            rows_v[0, i, pl.ds(j * 16, 16)] = jnp.zeros((16,), jnp.float32)
        return 0

    lax.fori_loop(0, CH, zero_body, 0)
    for off in range(0, SLICE, CH):
        nr = min(CH, SLICE - off)
        pltpu.sync_copy(rows_v.at[0, pl.ds(0, nr)],
                        acc_sp.at[pl.ds(s * SLICE + off, nr)])
    plsc.subcore_barrier()

    nch = jnp.where(c == 0, NCH0, NCH1)

    for b in range(NBUF):
        pltpu.async_copy(xs.at[src_v.at[b]], rows_v.at[b], gsem[b])

    def round_body(g, _):
        k0 = g * NBUF
        for b in range(NBUF):
            kb = k0 + b
            pltpu.make_async_copy(
                xs.at[src_v.at[kb]], rows_v.at[b], gsem[b]).wait()
            pltpu.sync_copy(rows_v.at[b], acc_sp.at[dst_v.at[kb]], add=True)

            @pl.when(kb + NBUF < nch)
            def _():
                pltpu.async_copy(
                    xs.at[src_v.at[kb + NBUF]], rows_v.at[b], gsem[b])
        return 0

    lax.fori_loop(0, nch // NBUF, round_body, 0)
    plsc.subcore_barrier()
    for off in range(0, SLICE, CH):
        nr = min(CH, SLICE - off)
        pltpu.sync_copy(acc_sp.at[pl.ds(s * SLICE + off, nr)],
                        out.at[c, pl.ds(s * SLICE + off, nr)])


def _dinv(deg_ref):
    deg = deg_ref[0, 0:N, 0:1] + deg_ref[1, 0:N, 0:1] + 1.0
    return lax.rsqrt(deg)


def _layer_norm(t, g, b):
    mu = jnp.mean(t, axis=-1, keepdims=True)
    var = jnp.mean((t - mu) ** 2, axis=-1, keepdims=True)
    return (t - mu) * lax.rsqrt(var + 1e-5) * g + b


def _tc1_body(x_ref, w_ref, b_ref, deg_ref, x0_ref, xs0_ref):
    x0 = jnp.maximum(
        jnp.dot(x_ref[...], w_ref[...], preferred_element_type=jnp.float32)
        + b_ref[...],
        0.0,
    )
    x0_ref[...] = x0
    xs0_ref[...] = x0 * _dinv(deg_ref)


def _tc2_body(pa_ref, xs0_ref, x0_ref, deg_ref, w1_ref, g1_ref, be1_ref,
              xs1_ref):
    dinv = _dinv(deg_ref)
    acc = pa_ref[0, 0:N, :] + pa_ref[1, 0:N, :]
    h = dinv * (acc + xs0_ref[...])
    mix = (1.0 - ALPHA) * h + ALPHA * x0_ref[...]
    t = jnp.dot(mix, w1_ref[...], preferred_element_type=jnp.float32)
    h1 = jnp.maximum(_layer_norm(t, g1_ref[...], be1_ref[...]), 0.0)
    xs1_ref[...] = h1 * dinv


def _tc3_body(pb_ref, xs1_ref, x0_ref, deg_ref, w2_ref, g2_ref, be2_ref,
              wr1_ref, br1_ref, gr_ref, ber_ref, wr2_ref, br2_ref, out_ref):
    dinv = _dinv(deg_ref)
    acc = pb_ref[0, 0:N, :] + pb_ref[1, 0:N, :]
    h = dinv * (acc + xs1_ref[...])
    mix = (1.0 - ALPHA) * h + ALPHA * x0_ref[...]
    t = jnp.dot(mix, w2_ref[...], preferred_element_type=jnp.float32)
    h2 = jnp.maximum(_layer_norm(t, g2_ref[...], be2_ref[...]), 0.0)
    zr = jnp.dot(h2, wr1_ref[...], preferred_element_type=jnp.float32) \
        + br1_ref[...]
    z = jnp.maximum(_layer_norm(zr, gr_ref[...], ber_ref[...]), 0.0)
    out_ref[...] = jnp.dot(
        z, wr2_ref[...], preferred_element_type=jnp.float32
    ) + br2_ref[...]


_tc1 = pl.pallas_call(
    _tc1_body,
    out_shape=(
        jax.ShapeDtypeStruct((N, H), jnp.float32),
        jax.ShapeDtypeStruct((N, H), jnp.float32),
    ),
)

_tc2 = pl.pallas_call(
    _tc2_body,
    out_shape=jax.ShapeDtypeStruct((N, H), jnp.float32),
)

_tc3 = pl.pallas_call(
    _tc3_body,
    out_shape=jax.ShapeDtypeStruct((N, OUT), jnp.float32),
)


def kernel(x, edge_index, edge_weight, W_in, b_in, W1, g1, be1, W2, g2, be2,
           Wr1, br1, gr, ber, Wr2, br2):
    src = edge_index[0]
    dst = edge_index[1]

    capd = NW * NCHD * CH
    dstd = jnp.concatenate(
        [dst, jnp.full((capd - E,), N, jnp.int32)]).reshape(NW, NCHD, CH)

    cap0 = NSUB * NCH0 * CH
    cap1 = NSUB * NCH1 * CH

    def split3(e, fill):
        ep = jnp.concatenate([e, jnp.full((cap0 + cap1 - E,), fill, e.dtype)])
        p0 = jnp.pad(ep[:cap0].reshape(NSUB, NCH0, CH),
                     ((0, 0), (0, NCHMAX - NCH0), (0, 0)),
                     constant_values=fill)
        p1 = jnp.pad(ep[cap0:].reshape(NSUB, NCH1, CH),
                     ((0, 0), (0, NCHMAX - NCH1), (0, 0)),
                     constant_values=fill)
        return jnp.concatenate([p0, p1], axis=0)

    src3 = split3(src, 0)
    dst3 = split3(dst, N)

    deg3 = _sc_deg(dstd)

    x0, xs0 = _tc1(x, W_in, b_in.reshape(1, H), deg3)
    pa = _sc_prop(xs0, src3, dst3)
    xs1 = _tc2(pa, xs0, x0, deg3, W1, g1.reshape(1, H), be1.reshape(1, H))
    pb = _sc_prop(xs1, src3, dst3)
    return _tc3(pb, xs1, x0, deg3, W2, g2.reshape(1, H), be2.reshape(1, H),
                Wr1, br1.reshape(1, 32), gr.reshape(1, 32),
                ber.reshape(1, 32), Wr2, br2.reshape(1, OUT))
